# Initial kernel scaffold; baseline (speedup 1.0000x reference)
import functools

import jax
import jax.numpy as jnp
from jax import lax
from jax.experimental import pallas as pl
from jax.experimental.pallas import tpu as pltpu
from jax.experimental.pallas import tpu_sc as plsc

N = 4096
E = 131072
D_IN = 300
DH = 64

NC = 2
NS = 16
C = 128
EPT = E // NS
NCHUNK = EPT // C
RPT = 264
R = NS * RPT
TRASH = N
OPT = N // NS

_PREC = lax.Precision.DEFAULT


def _make_proj(interpret=False):
  RB = 512

  def body(x_ref, wy_ref, wu_ref, tbl_ref, u_ref):
    x = x_ref[...]
    y = jnp.dot(x, wy_ref[...], preferred_element_type=jnp.float32,
                precision=_PREC)
    u = jnp.dot(x, wu_ref[...], preferred_element_type=jnp.float32,
                precision=_PREC)
    ones = jnp.ones((RB, 16), jnp.float32)
    tbl_ref[...] = jnp.concatenate([y, ones], axis=1)
    u_ref[...] = u

  grid = (N // RB, 2)
  return pl.pallas_call(
      body,
      grid=grid,
      in_specs=[
          pl.BlockSpec((RB, D_IN), lambda i, j: (i, 0)),
          pl.BlockSpec((D_IN, DH), lambda i, j: (0, j)),
          pl.BlockSpec((D_IN, DH), lambda i, j: (0, j)),
      ],
      out_specs=[
          pl.BlockSpec((RB, DH + 16), lambda i, j: (j * (N // RB) + i, 0)),
          pl.BlockSpec((RB, DH), lambda i, j: (i, j)),
      ],
      out_shape=[
          jax.ShapeDtypeStruct((2 * N, DH + 16), jnp.float32),
          jax.ShapeDtypeStruct((N, 2 * DH), jnp.float32),
      ],
      interpret=interpret,
  )


def _make_edges_prep(interpret=False):
  def body(e_ref, rows_ref, cols_ref):
    rows = e_ref[:, 0, :]
    cols = e_ref[:, 1, :]
    valid = rows != cols
    rows_ref[...] = jnp.where(valid, rows, TRASH)
    off = lax.broadcasted_iota(jnp.int32, (2, E), 0) * N
    cols_ref[...] = cols + off

  return pl.pallas_call(
      body,
      out_shape=[
          jax.ShapeDtypeStruct((2, E), jnp.int32),
          jax.ShapeDtypeStruct((2, E), jnp.int32),
      ],
      interpret=interpret,
  )


def _make_agg(width, interpret=False):
  mesh = plsc.VectorSubcoreMesh(core_axis_name="c", subcore_axis_name="s")

  @functools.partial(
      pl.kernel,
      out_type=jax.ShapeDtypeStruct((2, N, width), jnp.float32),
      mesh=mesh,
      scratch_types=[
          pltpu.VMEM((NCHUNK, C), jnp.int32),
          pltpu.VMEM((NCHUNK, C), jnp.int32),
          pltpu.VMEM((C, width), jnp.float32),
          pltpu.VMEM((RPT, width), jnp.float32),
          pltpu.VMEM_SHARED((R, width), jnp.float32),
          pltpu.SemaphoreType.DMA,
      ],
      interpret=interpret,
  )
  def agg(table, cols, rows, zz, out, colv, rowv, buf, rbuf, acc, sem):
    c = lax.axis_index("c")
    s = lax.axis_index("s")
    pltpu.sync_copy(zz.at[pl.ds(s * RPT, RPT)], rbuf)
    pltpu.sync_copy(rbuf, acc.at[pl.ds(s * RPT, RPT)])
    pltpu.sync_copy(cols.at[c, s], colv)
    pltpu.sync_copy(rows.at[c, s], rowv)
    plsc.subcore_barrier()

    def body(j, carry):
      pltpu.async_copy(table.at[colv.at[j]], buf, sem).wait()
      pltpu.sync_copy(buf, acc.at[rowv.at[j]], add=True)
      return carry

    lax.fori_loop(0, NCHUNK, body, 0)
    plsc.subcore_barrier()
    pltpu.sync_copy(acc.at[pl.ds(s * OPT, OPT)], rbuf.at[pl.ds(0, OPT)])
    pltpu.sync_copy(rbuf.at[pl.ds(0, OPT)], out.at[c, pl.ds(s * OPT, OPT)])

  return agg


def _make_h0(interpret=False):
  RB = 512

  def body(ss_ref, cnt_ref, u_ref, b_ref, out_ref):
    outs = []
    for sgn in range(2):
      cnt = cnt_ref[sgn][:, None]
      agg = ss_ref[sgn] / jnp.maximum(cnt, 1.0)
      pre = agg + u_ref[:, sgn * DH:(sgn + 1) * DH] + b_ref[sgn][None, :]
      nrm = jnp.sqrt(jnp.sum(pre * pre, axis=1, keepdims=True))
      outs.append(jnp.tanh(pre / jnp.maximum(nrm, 1e-12)))
    out_ref[...] = jnp.stack(outs)

  grid = (N // RB,)
  return pl.pallas_call(
      body,
      grid=grid,
      in_specs=[
          pl.BlockSpec((2, RB, DH), lambda i: (0, i, 0)),
          pl.BlockSpec((2, RB), lambda i: (0, i)),
          pl.BlockSpec((RB, 2 * DH), lambda i: (i, 0)),
          pl.BlockSpec((2, DH), lambda i: (0, 0)),
      ],
      out_specs=pl.BlockSpec((2, RB, DH), lambda i: (0, i, 0)),
      out_shape=jax.ShapeDtypeStruct((2, N, DH), jnp.float32),
      interpret=interpret,
  )


def _make_deep(interpret=False):
  RB = 512

  def body(sd_ref, h0_ref, cnt_ref, wd_ref, bd_ref, out_ref):
    a = []
    for sgn in range(2):
      cnt = cnt_ref[sgn][:, None]
      a.append((sd_ref[sgn] + h0_ref[sgn]) / (cnt + 1.0))
    hs = []
    for sgn in range(2):
      cat = jnp.concatenate([a[sgn], a[1 - sgn], h0_ref[sgn]], axis=1)
      pre = jnp.dot(cat, wd_ref[sgn], preferred_element_type=jnp.float32,
                    precision=_PREC) + bd_ref[sgn][None, :]
      nrm = jnp.sqrt(jnp.sum(pre * pre, axis=1, keepdims=True))
      hs.append(jnp.tanh(pre / jnp.maximum(nrm, 1e-12)))
    z = jnp.concatenate(hs, axis=1)
    nrm = jnp.sqrt(jnp.sum(z * z, axis=1, keepdims=True))
    out_ref[...] = z / jnp.maximum(nrm, 1e-12)

  grid = (N // RB,)
  return pl.pallas_call(
      body,
      grid=grid,
      in_specs=[
          pl.BlockSpec((2, RB, DH), lambda i: (0, i, 0)),
          pl.BlockSpec((2, RB, DH), lambda i: (0, i, 0)),
          pl.BlockSpec((2, RB), lambda i: (0, i)),
          pl.BlockSpec((2, 3 * DH, DH), lambda i: (0, 0, 0)),
          pl.BlockSpec((2, DH), lambda i: (0, 0)),
      ],
      out_specs=pl.BlockSpec((RB, 2 * DH), lambda i: (i, 0)),
      out_shape=jax.ShapeDtypeStruct((N, 2 * DH), jnp.float32),
      interpret=interpret,
  )


def _make_sim(interpret=False):
  BM, BN = 512, 1024
  GM, GN = N // BM, N // BN

  def body(a_ref, b_ref, m_ref, l_ref, pred_ref, ls_ref):
    i = pl.program_id(0)
    j = pl.program_id(1)
    g = lax.dot_general(a_ref[...], b_ref[...], (((1,), (1,)), ((), ())),
                        preferred_element_type=jnp.float32,
                        precision=_PREC)
    p = g * m_ref[...]
    pred_ref[...] = p
    d = p - l_ref[...]
    part = jnp.sum(d * d)

    @pl.when(jnp.logical_and(i == 0, j == 0))
    def _():
      ls_ref[0, 0] = 0.0

    ls_ref[0, 0] += part

    @pl.when(jnp.logical_and(i == GM - 1, j == GN - 1))
    def _():
      ls_ref[0, 0] = ls_ref[0, 0] / (N * N)

  return pl.pallas_call(
      body,
      grid=(GM, GN),
      in_specs=[
          pl.BlockSpec((BM, 2 * DH), lambda i, j: (i, 0)),
          pl.BlockSpec((BN, 2 * DH), lambda i, j: (j, 0)),
          pl.BlockSpec((BM, BN), lambda i, j: (i, j)),
          pl.BlockSpec((BM, BN), lambda i, j: (i, j)),
      ],
      out_specs=[
          pl.BlockSpec((BM, BN), lambda i, j: (i, j)),
          pl.BlockSpec((1, 1), lambda i, j: (0, 0),
                       memory_space=pltpu.SMEM),
      ],
      out_shape=[
          jax.ShapeDtypeStruct((N, N), jnp.float32),
          jax.ShapeDtypeStruct((1, 1), jnp.float32),
      ],
      interpret=interpret,
  )


_proj = _make_proj()
_edges_prep = _make_edges_prep()
_agg_base = _make_agg(DH + 16)
_agg_deep = _make_agg(DH)
_h0 = _make_h0()
_deep = _make_deep()
_sim = _make_sim()


def kernel(X, W_pos_base, b_pos_base, W_neg_base, b_neg_base, W_pos_deep,
           b_pos_deep, W_neg_deep, b_neg_deep, labels, label_mask,
           positive_edges, negative_edges):
  Wy = jnp.concatenate([W_pos_base[:D_IN], W_neg_base[:D_IN]], axis=1)
  Wu = jnp.concatenate([W_pos_base[D_IN:], W_neg_base[D_IN:]], axis=1)
  bcat = jnp.stack([b_pos_base, b_neg_base])
  Wd = jnp.stack([W_pos_deep, W_neg_deep])
  bd = jnp.stack([b_pos_deep, b_neg_deep])
  edges_all = jnp.stack([positive_edges, negative_edges]).astype(jnp.int32)

  table_base, U = _proj(X, Wy, Wu)
  rows_adj, cols_adj = _edges_prep(edges_all)
  cols_r = cols_adj.reshape(2, NS, NCHUNK, C)
  rows_r = rows_adj.reshape(2, NS, NCHUNK, C)

  zz80 = jnp.zeros((R, DH + 16), jnp.float32)
  s_base = _agg_base(table_base, cols_r, rows_r, zz80)
  ssum = s_base[:, :, :DH]
  cnt = s_base[:, :, DH]

  h0 = _h0(ssum, cnt, U, bcat)

  zz64 = jnp.zeros((R, DH), jnp.float32)
  s_deep = _agg_deep(h0.reshape(2 * N, DH), cols_r, rows_r, zz64)

  x_mol = _deep(s_deep, h0, cnt, Wd, bd)

  pred2d, lsum = _sim(x_mol, label_mask, labels.reshape(N, N))
  return (lsum[0, 0], x_mol, pred2d.reshape(-1))


# speedup vs baseline: 7.5727x; 7.5727x over previous
import functools

import jax
import jax.numpy as jnp
from jax import lax
from jax.experimental import pallas as pl
from jax.experimental.pallas import tpu as pltpu
from jax.experimental.pallas import tpu_sc as plsc

N = 4096
E = 131072
D_IN = 300
DH = 64

NC = 2
NS = 16
C = 128
EPT = E // NS
NCHUNK = EPT // C
RPT = 264
R = NS * RPT
TRASH = N
OPT = N // NS

_PREC = lax.Precision.DEFAULT


def _make_proj(interpret=False):
  RB = 512

  def body(x_ref, wy_ref, wu_ref, tbl_ref, u_ref):
    x = x_ref[...]
    y = jnp.dot(x, wy_ref[0], preferred_element_type=jnp.float32,
                precision=_PREC)
    u = jnp.dot(x, wu_ref[0], preferred_element_type=jnp.float32,
                precision=_PREC)
    ones = jnp.ones((RB, 16), jnp.float32)
    pad = jnp.zeros((RB, 48), jnp.float32)
    tbl_ref[...] = jnp.concatenate([y, ones, pad], axis=1)
    u_ref[...] = u[None]

  grid = (N // RB, 2)
  return pl.pallas_call(
      body,
      grid=grid,
      in_specs=[
          pl.BlockSpec((RB, D_IN), lambda i, j: (i, 0)),
          pl.BlockSpec((1, D_IN, DH), lambda i, j: (j, 0, 0)),
          pl.BlockSpec((1, D_IN, DH), lambda i, j: (j, 0, 0)),
      ],
      out_specs=[
          pl.BlockSpec((RB, 2 * DH), lambda i, j: (j * (N // RB) + i, 0)),
          pl.BlockSpec((1, RB, DH), lambda i, j: (j, i, 0)),
      ],
      out_shape=[
          jax.ShapeDtypeStruct((2 * N, 2 * DH), jnp.float32),
          jax.ShapeDtypeStruct((2, N, DH), jnp.float32),
      ],
      interpret=interpret,
  )


def _make_edges_prep(interpret=False):
  def body(e_ref, rows_ref, cols_ref):
    rows = e_ref[:, 0, :]
    cols = e_ref[:, 1, :]
    valid = rows != cols
    rows_ref[...] = jnp.where(valid, rows, TRASH)
    off = lax.broadcasted_iota(jnp.int32, (2, E), 0) * N
    cols_ref[...] = cols + off

  return pl.pallas_call(
      body,
      out_shape=[
          jax.ShapeDtypeStruct((2, E), jnp.int32),
          jax.ShapeDtypeStruct((2, E), jnp.int32),
      ],
      interpret=interpret,
  )


@functools.lru_cache(maxsize=None)
def _make_agg(width, interpret=False):
  mesh = plsc.VectorSubcoreMesh(core_axis_name="c", subcore_axis_name="s",
                                num_cores=NC, num_subcores=NS)

  @functools.partial(
      pl.kernel,
      out_type=jax.ShapeDtypeStruct((2, N, width), jnp.float32),
      mesh=mesh,
      scratch_types=[
          pltpu.VMEM((NCHUNK, C), jnp.int32),
          pltpu.VMEM((NCHUNK, C), jnp.int32),
          pltpu.VMEM((C, width), jnp.float32),
          pltpu.VMEM((RPT, width), jnp.float32),
          pltpu.VMEM_SHARED((R, width), jnp.float32),
          pltpu.SemaphoreType.DMA,
      ],
      interpret=interpret,
  )
  def agg(table, cols, rows, zz, out, colv, rowv, buf, rbuf, acc, sem):
    c = lax.axis_index("c")
    s = lax.axis_index("s")
    pltpu.sync_copy(zz.at[pl.ds(s * RPT, RPT)], rbuf)
    pltpu.sync_copy(rbuf, acc.at[pl.ds(s * RPT, RPT)])
    pltpu.sync_copy(cols.at[c, s], colv)
    pltpu.sync_copy(rows.at[c, s], rowv)
    plsc.subcore_barrier()

    def body(j, carry):
      pltpu.async_copy(table.at[colv.at[j]], buf, sem).wait()
      pltpu.sync_copy(buf, acc.at[rowv.at[j]], add=True)
      return carry

    lax.fori_loop(0, NCHUNK, body, 0)
    plsc.subcore_barrier()
    pltpu.sync_copy(acc.at[pl.ds(s * OPT, OPT)], rbuf.at[pl.ds(0, OPT)])
    pltpu.sync_copy(rbuf.at[pl.ds(0, OPT)], out.at[c, pl.ds(s * OPT, OPT)])

  return agg


def _make_h0(interpret=False):
  RB = 512

  def body(su_ref, u_ref, b_ref, wt_ref, h0_ref, t_ref):
    h0s, ts = [], []
    for sgn in range(2):
      su = su_ref[sgn]
      cnt = su[:, DH:DH + 1]
      agg = su[:, :DH] / jnp.maximum(cnt, 1.0)
      pre = agg + u_ref[sgn] + b_ref[sgn][None, :]
      nrm = jnp.sqrt(jnp.sum(pre * pre, axis=1, keepdims=True))
      h = jnp.tanh(pre / jnp.maximum(nrm, 1e-12))
      h0s.append(h)
      ts.append(jnp.dot(h, wt_ref[sgn], preferred_element_type=jnp.float32,
                        precision=_PREC))
    h0_ref[...] = jnp.stack(h0s)
    t_ref[...] = jnp.stack(ts)

  grid = (N // RB,)
  return pl.pallas_call(
      body,
      grid=grid,
      in_specs=[
          pl.BlockSpec((2, RB, 2 * DH), lambda i: (0, i, 0)),
          pl.BlockSpec((2, RB, DH), lambda i: (0, i, 0)),
          pl.BlockSpec((2, DH), lambda i: (0, 0)),
          pl.BlockSpec((2, DH, 2 * DH), lambda i: (0, 0, 0)),
      ],
      out_specs=[
          pl.BlockSpec((2, RB, DH), lambda i: (0, i, 0)),
          pl.BlockSpec((2, RB, 2 * DH), lambda i: (0, i, 0)),
      ],
      out_shape=[
          jax.ShapeDtypeStruct((2, N, DH), jnp.float32),
          jax.ShapeDtypeStruct((2, N, 2 * DH), jnp.float32),
      ],
      interpret=interpret,
  )


def _make_deep(interpret=False):
  RB = 512

  def body(sd_ref, t_ref, h0_ref, cnt_ref, ws_ref, bd_ref, out_ref):
    hs = []
    for sgn in range(2):
      oth = 1 - sgn
      c_own = cnt_ref[sgn][:, None] + 1.0
      c_oth = cnt_ref[oth][:, None] + 1.0
      own = (sd_ref[sgn][:, :DH] + t_ref[sgn][:, :DH]) / c_own
      cross = (sd_ref[oth][:, DH:] + t_ref[oth][:, DH:]) / c_oth
      slf = jnp.dot(h0_ref[sgn], ws_ref[sgn],
                    preferred_element_type=jnp.float32, precision=_PREC)
      pre = own + cross + slf + bd_ref[sgn][None, :]
      nrm = jnp.sqrt(jnp.sum(pre * pre, axis=1, keepdims=True))
      hs.append(jnp.tanh(pre / jnp.maximum(nrm, 1e-12)))
    z = jnp.concatenate(hs, axis=1)
    nrm = jnp.sqrt(jnp.sum(z * z, axis=1, keepdims=True))
    out_ref[...] = z / jnp.maximum(nrm, 1e-12)

  grid = (N // RB,)
  return pl.pallas_call(
      body,
      grid=grid,
      in_specs=[
          pl.BlockSpec((2, RB, 2 * DH), lambda i: (0, i, 0)),
          pl.BlockSpec((2, RB, 2 * DH), lambda i: (0, i, 0)),
          pl.BlockSpec((2, RB, DH), lambda i: (0, i, 0)),
          pl.BlockSpec((2, RB), lambda i: (0, i)),
          pl.BlockSpec((2, DH, DH), lambda i: (0, 0, 0)),
          pl.BlockSpec((2, DH), lambda i: (0, 0)),
      ],
      out_specs=pl.BlockSpec((RB, 2 * DH), lambda i: (i, 0)),
      out_shape=jax.ShapeDtypeStruct((N, 2 * DH), jnp.float32),
      interpret=interpret,
  )


def _make_sim(interpret=False):
  BM, BN = 512, 1024
  GM, GN = N // BM, N // BN

  def body(a_ref, b_ref, m_ref, l_ref, pred_ref, ls_ref):
    i = pl.program_id(0)
    j = pl.program_id(1)
    g = lax.dot_general(a_ref[...], b_ref[...], (((1,), (1,)), ((), ())),
                        preferred_element_type=jnp.float32,
                        precision=_PREC)
    p = g * m_ref[...]
    pred_ref[...] = p
    d = p - l_ref[...]
    part = jnp.sum(d * d)

    @pl.when(jnp.logical_and(i == 0, j == 0))
    def _():
      ls_ref[0, 0] = 0.0

    ls_ref[0, 0] += part

    @pl.when(jnp.logical_and(i == GM - 1, j == GN - 1))
    def _():
      ls_ref[0, 0] = ls_ref[0, 0] / (N * N)

  return pl.pallas_call(
      body,
      grid=(GM, GN),
      in_specs=[
          pl.BlockSpec((BM, 2 * DH), lambda i, j: (i, 0)),
          pl.BlockSpec((BN, 2 * DH), lambda i, j: (j, 0)),
          pl.BlockSpec((BM, BN), lambda i, j: (i, j)),
          pl.BlockSpec((BM, BN), lambda i, j: (i, j)),
      ],
      out_specs=[
          pl.BlockSpec((BM, BN), lambda i, j: (i, j)),
          pl.BlockSpec((1, 1), lambda i, j: (0, 0),
                       memory_space=pltpu.SMEM),
      ],
      out_shape=[
          jax.ShapeDtypeStruct((N, N), jnp.float32),
          jax.ShapeDtypeStruct((1, 1), jnp.float32),
      ],
      interpret=interpret,
  )


_proj = _make_proj()
_edges_prep = _make_edges_prep()
_h0 = _make_h0()
_deep = _make_deep()
_sim = _make_sim()


def kernel(X, W_pos_base, b_pos_base, W_neg_base, b_neg_base, W_pos_deep,
           b_pos_deep, W_neg_deep, b_neg_deep, labels, label_mask,
           positive_edges, negative_edges):
  Wy = jnp.stack([W_pos_base[:D_IN], W_neg_base[:D_IN]])
  Wu = jnp.stack([W_pos_base[D_IN:], W_neg_base[D_IN:]])
  bcat = jnp.stack([b_pos_base, b_neg_base])
  Wt = jnp.stack([
      jnp.concatenate([W_pos_deep[:DH], W_neg_deep[DH:2 * DH]], axis=1),
      jnp.concatenate([W_neg_deep[:DH], W_pos_deep[DH:2 * DH]], axis=1),
  ])
  Wself = jnp.stack([W_pos_deep[2 * DH:], W_neg_deep[2 * DH:]])
  bd = jnp.stack([b_pos_deep, b_neg_deep])
  edges_all = jnp.stack([positive_edges, negative_edges]).astype(jnp.int32)

  table_base, U = _proj(X, Wy, Wu)
  rows_adj, cols_adj = _edges_prep(edges_all)
  cols_r = cols_adj.reshape(2, NS, NCHUNK, C)
  rows_r = rows_adj.reshape(2, NS, NCHUNK, C)

  zz = jnp.zeros((R, 2 * DH), jnp.float32)
  agg = _make_agg(2 * DH)
  s_base = agg(table_base, cols_r, rows_r, zz)
  cnt = s_base[:, :, DH]

  h0, t_deep = _h0(s_base, U, bcat, Wt)

  s_deep = agg(t_deep.reshape(2 * N, 2 * DH), cols_r, rows_r, zz)

  x_mol = _deep(s_deep, t_deep, h0, cnt, Wself, bd)

  pred2d, lsum = _sim(x_mol, x_mol, label_mask, labels.reshape(N, N))
  return (lsum[0, 0], x_mol, pred2d.reshape(-1))


# speedup vs baseline: 10.8731x; 1.4358x over previous
import functools

import jax
import jax.numpy as jnp
from jax import lax
from jax.experimental import pallas as pl
from jax.experimental.pallas import tpu as pltpu
from jax.experimental.pallas import tpu_sc as plsc

N = 4096
E = 131072
D_IN = 300
DH = 64

NC = 2
NS = 16
C = 128
EPT = E // NS
NCHUNK = EPT // C
RPT = 264
R = NS * RPT
TRASH = N
OPT = N // NS

_PREC = lax.Precision.DEFAULT


def _make_proj(interpret=False):
  RB = 512

  def body(x_ref, wy_ref, wu_ref, tbl_ref, u_ref):
    x = x_ref[...]
    y = jnp.dot(x, wy_ref[0], preferred_element_type=jnp.float32,
                precision=_PREC)
    u = jnp.dot(x, wu_ref[0], preferred_element_type=jnp.float32,
                precision=_PREC)
    ones = jnp.ones((RB, 16), jnp.float32)
    pad = jnp.zeros((RB, 48), jnp.float32)
    tbl_ref[...] = jnp.concatenate([y, ones, pad], axis=1)
    u_ref[...] = u[None]

  grid = (N // RB, 2)
  return pl.pallas_call(
      body,
      grid=grid,
      in_specs=[
          pl.BlockSpec((RB, D_IN), lambda i, j: (i, 0)),
          pl.BlockSpec((1, D_IN, DH), lambda i, j: (j, 0, 0)),
          pl.BlockSpec((1, D_IN, DH), lambda i, j: (j, 0, 0)),
      ],
      out_specs=[
          pl.BlockSpec((RB, 2 * DH), lambda i, j: (j * (N // RB) + i, 0)),
          pl.BlockSpec((1, RB, DH), lambda i, j: (j, i, 0)),
      ],
      out_shape=[
          jax.ShapeDtypeStruct((2 * N, 2 * DH), jnp.float32),
          jax.ShapeDtypeStruct((2, N, DH), jnp.float32),
      ],
      interpret=interpret,
  )


def _make_edges_prep(interpret=False):
  def body(e_ref, rows_ref, cols_ref):
    rows = e_ref[:, 0, :]
    cols = e_ref[:, 1, :]
    valid = rows != cols
    rows_ref[...] = jnp.where(valid, rows, TRASH)
    off = lax.broadcasted_iota(jnp.int32, (2, E), 0) * N
    cols_ref[...] = cols + off

  return pl.pallas_call(
      body,
      out_shape=[
          jax.ShapeDtypeStruct((2, E), jnp.int32),
          jax.ShapeDtypeStruct((2, E), jnp.int32),
      ],
      interpret=interpret,
  )


@functools.lru_cache(maxsize=None)
def _make_agg(width, interpret=False):
  mesh = plsc.VectorSubcoreMesh(core_axis_name="c", subcore_axis_name="s",
                                num_cores=NC, num_subcores=NS)

  @functools.partial(
      pl.kernel,
      out_type=jax.ShapeDtypeStruct((2, N, width), jnp.float32),
      mesh=mesh,
      scratch_types=[
          pltpu.VMEM((NCHUNK, C), jnp.int32),
          pltpu.VMEM((NCHUNK, C), jnp.int32),
          pltpu.VMEM((2, C, width), jnp.float32),
          pltpu.VMEM((RPT, width), jnp.float32),
          pltpu.VMEM_SHARED((R, width), jnp.float32),
          pltpu.SemaphoreType.DMA((2,)),
          pltpu.SemaphoreType.DMA,
      ],
      interpret=interpret,
  )
  def agg(table, cols, rows, zz, out, colv, rowv, buf, rbuf, acc, gsem, ssem):
    c = lax.axis_index("c")
    s = lax.axis_index("s")
    pltpu.sync_copy(zz.at[pl.ds(s * RPT, RPT)], rbuf)
    pltpu.sync_copy(rbuf, acc.at[pl.ds(s * RPT, RPT)])
    pltpu.sync_copy(cols.at[c, s], colv)
    pltpu.sync_copy(rows.at[c, s], rowv)
    plsc.subcore_barrier()

    pltpu.async_copy(table.at[colv.at[0]], buf.at[0], gsem.at[0])

    def body(j, carry):
      slot = lax.rem(j, 2)
      nslot = lax.rem(j + 1, 2)

      @pl.when(j >= 1)
      def _():
        pltpu.make_async_copy(
            buf.at[nslot], acc.at[rowv.at[j - 1]], ssem).wait()

      @pl.when(j < NCHUNK - 1)
      def _():
        pltpu.async_copy(table.at[colv.at[j + 1]], buf.at[nslot],
                         gsem.at[nslot])

      pltpu.make_async_copy(table.at[colv.at[j]], buf.at[slot],
                            gsem.at[slot]).wait()
      pltpu.async_copy(buf.at[slot], acc.at[rowv.at[j]], ssem, add=True)
      return carry

    lax.fori_loop(0, NCHUNK, body, 0)
    last = NCHUNK - 1
    pltpu.make_async_copy(
        buf.at[lax.rem(last, 2)], acc.at[rowv.at[last]], ssem).wait()
    plsc.subcore_barrier()
    pltpu.sync_copy(acc.at[pl.ds(s * OPT, OPT)], rbuf.at[pl.ds(0, OPT)])
    pltpu.sync_copy(rbuf.at[pl.ds(0, OPT)], out.at[c, pl.ds(s * OPT, OPT)])

  return agg


def _make_h0(interpret=False):
  RB = 512

  def body(su_ref, u_ref, b_ref, wt_ref, h0_ref, t_ref):
    h0s, ts = [], []
    for sgn in range(2):
      su = su_ref[sgn]
      cnt = su[:, DH:DH + 1]
      agg = su[:, :DH] / jnp.maximum(cnt, 1.0)
      pre = agg + u_ref[sgn] + b_ref[sgn][None, :]
      nrm = jnp.sqrt(jnp.sum(pre * pre, axis=1, keepdims=True))
      h = jnp.tanh(pre / jnp.maximum(nrm, 1e-12))
      h0s.append(h)
      ts.append(jnp.dot(h, wt_ref[sgn], preferred_element_type=jnp.float32,
                        precision=_PREC))
    h0_ref[...] = jnp.stack(h0s)
    t_ref[...] = jnp.stack(ts)

  grid = (N // RB,)
  return pl.pallas_call(
      body,
      grid=grid,
      in_specs=[
          pl.BlockSpec((2, RB, 2 * DH), lambda i: (0, i, 0)),
          pl.BlockSpec((2, RB, DH), lambda i: (0, i, 0)),
          pl.BlockSpec((2, DH), lambda i: (0, 0)),
          pl.BlockSpec((2, DH, 2 * DH), lambda i: (0, 0, 0)),
      ],
      out_specs=[
          pl.BlockSpec((2, RB, DH), lambda i: (0, i, 0)),
          pl.BlockSpec((2, RB, 2 * DH), lambda i: (0, i, 0)),
      ],
      out_shape=[
          jax.ShapeDtypeStruct((2, N, DH), jnp.float32),
          jax.ShapeDtypeStruct((2, N, 2 * DH), jnp.float32),
      ],
      interpret=interpret,
  )


def _make_deep(interpret=False):
  RB = 512

  def body(sd_ref, t_ref, h0_ref, cnt_ref, ws_ref, bd_ref, out_ref):
    hs = []
    for sgn in range(2):
      oth = 1 - sgn
      c_own = cnt_ref[sgn][:, None] + 1.0
      c_oth = cnt_ref[oth][:, None] + 1.0
      own = (sd_ref[sgn][:, :DH] + t_ref[sgn][:, :DH]) / c_own
      cross = (sd_ref[oth][:, DH:] + t_ref[oth][:, DH:]) / c_oth
      slf = jnp.dot(h0_ref[sgn], ws_ref[sgn],
                    preferred_element_type=jnp.float32, precision=_PREC)
      pre = own + cross + slf + bd_ref[sgn][None, :]
      nrm = jnp.sqrt(jnp.sum(pre * pre, axis=1, keepdims=True))
      hs.append(jnp.tanh(pre / jnp.maximum(nrm, 1e-12)))
    z = jnp.concatenate(hs, axis=1)
    nrm = jnp.sqrt(jnp.sum(z * z, axis=1, keepdims=True))
    out_ref[...] = z / jnp.maximum(nrm, 1e-12)

  grid = (N // RB,)
  return pl.pallas_call(
      body,
      grid=grid,
      in_specs=[
          pl.BlockSpec((2, RB, 2 * DH), lambda i: (0, i, 0)),
          pl.BlockSpec((2, RB, 2 * DH), lambda i: (0, i, 0)),
          pl.BlockSpec((2, RB, DH), lambda i: (0, i, 0)),
          pl.BlockSpec((2, RB), lambda i: (0, i)),
          pl.BlockSpec((2, DH, DH), lambda i: (0, 0, 0)),
          pl.BlockSpec((2, DH), lambda i: (0, 0)),
      ],
      out_specs=pl.BlockSpec((RB, 2 * DH), lambda i: (i, 0)),
      out_shape=jax.ShapeDtypeStruct((N, 2 * DH), jnp.float32),
      interpret=interpret,
  )


def _make_sim(interpret=False):
  BM, BN = 512, 1024
  GM, GN = N // BM, N // BN

  def body(a_ref, b_ref, m_ref, l_ref, pred_ref, ls_ref):
    i = pl.program_id(0)
    j = pl.program_id(1)
    g = lax.dot_general(a_ref[...], b_ref[...], (((1,), (1,)), ((), ())),
                        preferred_element_type=jnp.float32,
                        precision=_PREC)
    p = g * m_ref[...]
    pred_ref[...] = p.reshape(BM, BN // 128, 128)
    d = p - l_ref[...]
    part = jnp.sum(d * d)

    @pl.when(jnp.logical_and(i == 0, j == 0))
    def _():
      ls_ref[0, 0] = 0.0

    ls_ref[0, 0] += part

    @pl.when(jnp.logical_and(i == GM - 1, j == GN - 1))
    def _():
      ls_ref[0, 0] = ls_ref[0, 0] / (N * N)

  return pl.pallas_call(
      body,
      grid=(GM, GN),
      in_specs=[
          pl.BlockSpec((BM, 2 * DH), lambda i, j: (i, 0)),
          pl.BlockSpec((BN, 2 * DH), lambda i, j: (j, 0)),
          pl.BlockSpec((BM, BN), lambda i, j: (i, j)),
          pl.BlockSpec((BM, BN), lambda i, j: (i, j)),
      ],
      out_specs=[
          pl.BlockSpec((BM, BN // 128, 128), lambda i, j: (i, j, 0)),
          pl.BlockSpec((1, 1), lambda i, j: (0, 0),
                       memory_space=pltpu.SMEM),
      ],
      out_shape=[
          jax.ShapeDtypeStruct((N, N // 128, 128), jnp.float32),
          jax.ShapeDtypeStruct((1, 1), jnp.float32),
      ],
      interpret=interpret,
  )


_proj = _make_proj()
_edges_prep = _make_edges_prep()
_h0 = _make_h0()
_deep = _make_deep()
_sim = _make_sim()


def kernel(X, W_pos_base, b_pos_base, W_neg_base, b_neg_base, W_pos_deep,
           b_pos_deep, W_neg_deep, b_neg_deep, labels, label_mask,
           positive_edges, negative_edges):
  Wy = jnp.stack([W_pos_base[:D_IN], W_neg_base[:D_IN]])
  Wu = jnp.stack([W_pos_base[D_IN:], W_neg_base[D_IN:]])
  bcat = jnp.stack([b_pos_base, b_neg_base])
  Wt = jnp.stack([
      jnp.concatenate([W_pos_deep[:DH], W_neg_deep[DH:2 * DH]], axis=1),
      jnp.concatenate([W_neg_deep[:DH], W_pos_deep[DH:2 * DH]], axis=1),
  ])
  Wself = jnp.stack([W_pos_deep[2 * DH:], W_neg_deep[2 * DH:]])
  bd = jnp.stack([b_pos_deep, b_neg_deep])
  edges_all = jnp.stack([positive_edges, negative_edges]).astype(jnp.int32)

  table_base, U = _proj(X, Wy, Wu)
  rows_adj, cols_adj = _edges_prep(edges_all)
  cols_r = cols_adj.reshape(2, NS, NCHUNK, C)
  rows_r = rows_adj.reshape(2, NS, NCHUNK, C)

  zz = jnp.zeros((R, 2 * DH), jnp.float32)
  agg = _make_agg(2 * DH)
  s_base = agg(table_base, cols_r, rows_r, zz)
  cnt = s_base[:, :, DH]

  h0, t_deep = _h0(s_base, U, bcat, Wt)

  s_deep = agg(t_deep.reshape(2 * N, 2 * DH), cols_r, rows_r, zz)

  x_mol = _deep(s_deep, t_deep, h0, cnt, Wself, bd)

  pred2d, lsum = _sim(x_mol, x_mol, label_mask, labels.reshape(N, N))
  return (lsum[0, 0], x_mol, pred2d.reshape(-1))


# speedup vs baseline: 11.2057x; 1.0306x over previous
import functools

import jax
import jax.numpy as jnp
from jax import lax
from jax.experimental import pallas as pl
from jax.experimental.pallas import tpu as pltpu
from jax.experimental.pallas import tpu_sc as plsc

N = 4096
E = 131072
D_IN = 300
DH = 64

NC = 2
NS = 16
C = 128
NBUF = 4
EPT = E // NS
NCHUNK = EPT // C
RPT = 264
R = NS * RPT
TRASH = N
OPT = N // NS

_PREC = lax.Precision.DEFAULT


def _make_proj(interpret=False):
  RB = 512

  def body(x_ref, wy_ref, wu_ref, e_ref, tbl_ref, u_ref, rows_ref, cols_ref):
    x = x_ref[...]
    y = jnp.dot(x, wy_ref[0], preferred_element_type=jnp.float32,
                precision=_PREC)
    u = jnp.dot(x, wu_ref[0], preferred_element_type=jnp.float32,
                precision=_PREC)
    ones = jnp.ones((RB, 16), jnp.float32)
    pad = jnp.zeros((RB, 48), jnp.float32)
    tbl_ref[...] = jnp.concatenate([y, ones, pad], axis=1)
    u_ref[...] = u[None]

    @pl.when(jnp.logical_and(pl.program_id(0) == 0, pl.program_id(1) == 0))
    def _():
      rows = e_ref[:, 0, :]
      cols = e_ref[:, 1, :]
      valid = rows != cols
      rows_ref[...] = jnp.where(valid, rows, TRASH)
      off = lax.broadcasted_iota(jnp.int32, (2, E), 0) * N
      cols_ref[...] = cols + off

  grid = (N // RB, 2)
  return pl.pallas_call(
      body,
      grid=grid,
      in_specs=[
          pl.BlockSpec((RB, D_IN), lambda i, j: (i, 0)),
          pl.BlockSpec((1, D_IN, DH), lambda i, j: (j, 0, 0)),
          pl.BlockSpec((1, D_IN, DH), lambda i, j: (j, 0, 0)),
          pl.BlockSpec((2, 2, E), lambda i, j: (0, 0, 0)),
      ],
      out_specs=[
          pl.BlockSpec((RB, 2 * DH), lambda i, j: (j * (N // RB) + i, 0)),
          pl.BlockSpec((1, RB, DH), lambda i, j: (j, i, 0)),
          pl.BlockSpec((2, E), lambda i, j: (0, 0)),
          pl.BlockSpec((2, E), lambda i, j: (0, 0)),
      ],
      out_shape=[
          jax.ShapeDtypeStruct((2 * N, 2 * DH), jnp.float32),
          jax.ShapeDtypeStruct((2, N, DH), jnp.float32),
          jax.ShapeDtypeStruct((2, E), jnp.int32),
          jax.ShapeDtypeStruct((2, E), jnp.int32),
      ],
      interpret=interpret,
  )


@functools.lru_cache(maxsize=None)
def _make_agg(width, interpret=False):
  mesh = plsc.VectorSubcoreMesh(core_axis_name="c", subcore_axis_name="s",
                                num_cores=NC, num_subcores=NS)

  @functools.partial(
      pl.kernel,
      out_type=jax.ShapeDtypeStruct((2, N, width), jnp.float32),
      mesh=mesh,
      scratch_types=[
          pltpu.VMEM((NCHUNK, C), jnp.int32),
          pltpu.VMEM((NCHUNK, C), jnp.int32),
          pltpu.VMEM((NBUF, C, width), jnp.float32),
          pltpu.VMEM_SHARED((R, width), jnp.float32),
          pltpu.SemaphoreType.DMA((NBUF,)),
          pltpu.SemaphoreType.DMA((NBUF,)),
      ],
      interpret=interpret,
  )
  def agg(table, cols, rows, zz, out, colv, rowv, buf, acc, gsem, ssem):
    c = lax.axis_index("c")
    s = lax.axis_index("s")
    pltpu.sync_copy(zz.at[pl.ds(s * RPT, RPT)], acc.at[pl.ds(s * RPT, RPT)])
    pltpu.sync_copy(cols.at[c, s], colv)
    pltpu.sync_copy(rows.at[c, s], rowv)
    plsc.subcore_barrier()

    for p in range(NBUF - 1):
      pltpu.async_copy(table.at[colv.at[p]], buf.at[p], gsem.at[p])

    def body(j, carry):
      slot = lax.rem(j, NBUF)
      pslot = lax.rem(j + NBUF - 1, NBUF)

      @pl.when(j + NBUF - 1 < NCHUNK)
      def _():
        @pl.when(j >= 1)
        def _():
          pltpu.make_async_copy(
              buf.at[pslot], acc.at[rowv.at[j - 1]], ssem.at[pslot]).wait()

        pltpu.async_copy(table.at[colv.at[j + NBUF - 1]], buf.at[pslot],
                         gsem.at[pslot])

      pltpu.make_async_copy(table.at[colv.at[j]], buf.at[slot],
                            gsem.at[slot]).wait()
      pltpu.async_copy(buf.at[slot], acc.at[rowv.at[j]], ssem.at[slot],
                       add=True)
      return carry

    lax.fori_loop(0, NCHUNK, body, 0)
    for p in range(NBUF):
      last = NCHUNK - NBUF + p
      pltpu.make_async_copy(
          buf.at[last % NBUF], acc.at[rowv.at[last]],
          ssem.at[last % NBUF]).wait()
    plsc.subcore_barrier()
    pltpu.sync_copy(acc.at[pl.ds(s * OPT, OPT)], out.at[c, pl.ds(s * OPT, OPT)])

  return agg


def _make_h0(interpret=False):
  RB = 512

  def body(su_ref, u_ref, b_ref, wt_ref, h0_ref, t_ref):
    su = su_ref[0]
    cnt = su[:, DH:DH + 1]
    agg = su[:, :DH] / jnp.maximum(cnt, 1.0)
    pre = agg + u_ref[0] + b_ref[0]
    nrm = jnp.sqrt(jnp.sum(pre * pre, axis=1, keepdims=True))
    h = jnp.tanh(pre / jnp.maximum(nrm, 1e-12))
    h0_ref[...] = h[None]
    t_ref[...] = jnp.dot(h, wt_ref[0], preferred_element_type=jnp.float32,
                         precision=_PREC)

  grid = (N // RB, 2)
  return pl.pallas_call(
      body,
      grid=grid,
      in_specs=[
          pl.BlockSpec((1, RB, 2 * DH), lambda i, j: (j, i, 0)),
          pl.BlockSpec((1, RB, DH), lambda i, j: (j, i, 0)),
          pl.BlockSpec((1, 1, DH), lambda i, j: (j, 0, 0)),
          pl.BlockSpec((1, DH, 2 * DH), lambda i, j: (j, 0, 0)),
      ],
      out_specs=[
          pl.BlockSpec((1, RB, DH), lambda i, j: (j, i, 0)),
          pl.BlockSpec((RB, 2 * DH), lambda i, j: (j * (N // RB) + i, 0)),
      ],
      out_shape=[
          jax.ShapeDtypeStruct((2, N, DH), jnp.float32),
          jax.ShapeDtypeStruct((2 * N, 2 * DH), jnp.float32),
      ],
      interpret=interpret,
  )


def _make_deep(interpret=False):
  RB = 512

  def body(sd_ref, tp_ref, tn_ref, h0_ref, cnt_ref, ws_ref, bd_ref, out_ref):
    ts = [tp_ref[...], tn_ref[...]]
    hs = []
    for sgn in range(2):
      oth = 1 - sgn
      c_own = cnt_ref[sgn][:, None] + 1.0
      c_oth = cnt_ref[oth][:, None] + 1.0
      own = (sd_ref[sgn][:, :DH] + ts[sgn][:, :DH]) / c_own
      cross = (sd_ref[oth][:, DH:] + ts[oth][:, DH:]) / c_oth
      slf = jnp.dot(h0_ref[sgn], ws_ref[sgn],
                    preferred_element_type=jnp.float32, precision=_PREC)
      pre = own + cross + slf + bd_ref[sgn][None, :]
      nrm = jnp.sqrt(jnp.sum(pre * pre, axis=1, keepdims=True))
      hs.append(jnp.tanh(pre / jnp.maximum(nrm, 1e-12)))
    z = jnp.concatenate(hs, axis=1)
    nrm = jnp.sqrt(jnp.sum(z * z, axis=1, keepdims=True))
    out_ref[...] = z / jnp.maximum(nrm, 1e-12)

  grid = (N // RB,)
  return pl.pallas_call(
      body,
      grid=grid,
      in_specs=[
          pl.BlockSpec((2, RB, 2 * DH), lambda i: (0, i, 0)),
          pl.BlockSpec((RB, 2 * DH), lambda i: (i, 0)),
          pl.BlockSpec((RB, 2 * DH), lambda i: (N // RB + i, 0)),
          pl.BlockSpec((2, RB, DH), lambda i: (0, i, 0)),
          pl.BlockSpec((2, RB), lambda i: (0, i)),
          pl.BlockSpec((2, DH, DH), lambda i: (0, 0, 0)),
          pl.BlockSpec((2, DH), lambda i: (0, 0)),
      ],
      out_specs=pl.BlockSpec((RB, 2 * DH), lambda i: (i, 0)),
      out_shape=jax.ShapeDtypeStruct((N, 2 * DH), jnp.float32),
      interpret=interpret,
  )


def _make_sim(interpret=False):
  BM, BN = 512, 2048
  GM, GN = N // BM, N // BN

  def body(a_ref, b_ref, m_ref, l_ref, pred_ref, ls_ref):
    i = pl.program_id(0)
    j = pl.program_id(1)
    g = lax.dot_general(a_ref[...], b_ref[...], (((1,), (1,)), ((), ())),
                        preferred_element_type=jnp.float32,
                        precision=_PREC)
    p = g * m_ref[...]
    pred_ref[...] = p.reshape(BM, BN // 128, 128)
    d = p - l_ref[...]
    part = jnp.sum(d * d)

    @pl.when(jnp.logical_and(i == 0, j == 0))
    def _():
      ls_ref[0, 0] = 0.0

    ls_ref[0, 0] += part

    @pl.when(jnp.logical_and(i == GM - 1, j == GN - 1))
    def _():
      ls_ref[0, 0] = ls_ref[0, 0] / (N * N)

  return pl.pallas_call(
      body,
      grid=(GM, GN),
      in_specs=[
          pl.BlockSpec((BM, 2 * DH), lambda i, j: (i, 0)),
          pl.BlockSpec((BN, 2 * DH), lambda i, j: (j, 0)),
          pl.BlockSpec((BM, BN), lambda i, j: (i, j)),
          pl.BlockSpec((BM, BN), lambda i, j: (i, j)),
      ],
      out_specs=[
          pl.BlockSpec((BM, BN // 128, 128), lambda i, j: (i, j, 0)),
          pl.BlockSpec((1, 1), lambda i, j: (0, 0),
                       memory_space=pltpu.SMEM),
      ],
      out_shape=[
          jax.ShapeDtypeStruct((N, N // 128, 128), jnp.float32),
          jax.ShapeDtypeStruct((1, 1), jnp.float32),
      ],
      interpret=interpret,
  )


_proj = _make_proj()
_h0 = _make_h0()
_deep = _make_deep()
_sim = _make_sim()


def kernel(X, W_pos_base, b_pos_base, W_neg_base, b_neg_base, W_pos_deep,
           b_pos_deep, W_neg_deep, b_neg_deep, labels, label_mask,
           positive_edges, negative_edges):
  Wy = jnp.stack([W_pos_base[:D_IN], W_neg_base[:D_IN]])
  Wu = jnp.stack([W_pos_base[D_IN:], W_neg_base[D_IN:]])
  bcat = jnp.stack([b_pos_base, b_neg_base])
  Wt = jnp.stack([
      jnp.concatenate([W_pos_deep[:DH], W_neg_deep[DH:2 * DH]], axis=1),
      jnp.concatenate([W_neg_deep[:DH], W_pos_deep[DH:2 * DH]], axis=1),
  ])
  Wself = jnp.stack([W_pos_deep[2 * DH:], W_neg_deep[2 * DH:]])
  bd = jnp.stack([b_pos_deep, b_neg_deep])
  edges_all = jnp.stack([positive_edges, negative_edges]).astype(jnp.int32)

  table_base, U, rows_adj, cols_adj = _proj(X, Wy, Wu, edges_all)
  cols_r = cols_adj.reshape(2, NS, NCHUNK, C)
  rows_r = rows_adj.reshape(2, NS, NCHUNK, C)

  zz = jnp.zeros((R, 2 * DH), jnp.float32)
  agg = _make_agg(2 * DH)
  s_base = agg(table_base, cols_r, rows_r, zz)
  cnt = s_base[:, :, DH]

  h0, t_deep = _h0(s_base, U, bcat[:, None, :], Wt)

  s_deep = agg(t_deep, cols_r, rows_r, zz)

  x_mol = _deep(s_deep, t_deep, t_deep, h0, cnt, Wself, bd)

  pred2d, lsum = _sim(x_mol, x_mol, label_mask, labels.reshape(N, N))
  return (lsum[0, 0], x_mol, pred2d.reshape(-1))


# speedup vs baseline: 11.3854x; 1.0160x over previous
import functools

import jax
import jax.numpy as jnp
from jax import lax
from jax.experimental import pallas as pl
from jax.experimental.pallas import tpu as pltpu
from jax.experimental.pallas import tpu_sc as plsc

N = 4096
E = 131072
D_IN = 300
DH = 64

NC = 2
NS = 16
C = 128
NBUF = 4
EPT = E // NS
NCHUNK = EPT // C
RPT = 264
R = NS * RPT
TRASH = N
OPT = N // NS

_PREC = lax.Precision.DEFAULT


def _make_proj(interpret=False):
  RB = 512

  def body(x_ref, wy_ref, wu_ref, e_ref, tbl_ref, u_ref, rows_ref, cols_ref):
    x = x_ref[...]
    y = jnp.dot(x, wy_ref[0], preferred_element_type=jnp.float32,
                precision=_PREC)
    u = jnp.dot(x, wu_ref[0], preferred_element_type=jnp.float32,
                precision=_PREC)
    ones = jnp.ones((RB, 16), jnp.float32)
    pad = jnp.zeros((RB, 48), jnp.float32)
    tbl_ref[...] = jnp.concatenate([y, ones, pad], axis=1)
    u_ref[...] = u[None]

    @pl.when(jnp.logical_and(pl.program_id(0) == 0, pl.program_id(1) == 0))
    def _():
      rows = e_ref[:, 0, :]
      cols = e_ref[:, 1, :]
      valid = rows != cols
      rows_ref[...] = jnp.where(valid, rows, TRASH)
      off = lax.broadcasted_iota(jnp.int32, (2, E), 0) * N
      cols_ref[...] = cols + off

  grid = (N // RB, 2)
  return pl.pallas_call(
      body,
      grid=grid,
      in_specs=[
          pl.BlockSpec((RB, D_IN), lambda i, j: (i, 0)),
          pl.BlockSpec((1, D_IN, DH), lambda i, j: (j, 0, 0)),
          pl.BlockSpec((1, D_IN, DH), lambda i, j: (j, 0, 0)),
          pl.BlockSpec((2, 2, E), lambda i, j: (0, 0, 0)),
      ],
      out_specs=[
          pl.BlockSpec((RB, 2 * DH), lambda i, j: (j * (N // RB) + i, 0)),
          pl.BlockSpec((1, RB, DH), lambda i, j: (j, i, 0)),
          pl.BlockSpec((2, E), lambda i, j: (0, 0)),
          pl.BlockSpec((2, E), lambda i, j: (0, 0)),
      ],
      out_shape=[
          jax.ShapeDtypeStruct((2 * N, 2 * DH), jnp.float32),
          jax.ShapeDtypeStruct((2, N, DH), jnp.float32),
          jax.ShapeDtypeStruct((2, E), jnp.int32),
          jax.ShapeDtypeStruct((2, E), jnp.int32),
      ],
      interpret=interpret,
  )


@functools.lru_cache(maxsize=None)
def _make_agg(width, interpret=False):
  mesh = plsc.VectorSubcoreMesh(core_axis_name="c", subcore_axis_name="s",
                                num_cores=NC, num_subcores=NS)

  @functools.partial(
      pl.kernel,
      out_type=jax.ShapeDtypeStruct((2, N, width), jnp.float32),
      mesh=mesh,
      scratch_types=[
          pltpu.VMEM((NCHUNK, C), jnp.int32),
          pltpu.VMEM((NCHUNK, C), jnp.int32),
          pltpu.VMEM((NBUF, C, width), jnp.float32),
          pltpu.VMEM_SHARED((R, width), jnp.float32),
          pltpu.SemaphoreType.DMA((NBUF,)),
          pltpu.SemaphoreType.DMA((NBUF,)),
      ],
      interpret=interpret,
  )
  def agg(table, cols, rows, zz, out, colv, rowv, buf, acc, gsem, ssem):
    c = lax.axis_index("c")
    s = lax.axis_index("s")
    pltpu.sync_copy(zz.at[pl.ds(s * RPT, RPT)], acc.at[pl.ds(s * RPT, RPT)])
    pltpu.sync_copy(cols.at[c, s], colv)
    pltpu.sync_copy(rows.at[c, s], rowv)
    plsc.subcore_barrier()

    for p in range(NBUF - 1):
      pltpu.async_copy(table.at[colv.at[p]], buf.at[p], gsem.at[p])

    def body(j, carry):
      slot = lax.rem(j, NBUF)
      pslot = lax.rem(j + NBUF - 1, NBUF)

      @pl.when(j + NBUF - 1 < NCHUNK)
      def _():
        @pl.when(j >= 1)
        def _():
          pltpu.make_async_copy(
              buf.at[pslot], acc.at[rowv.at[j - 1]], ssem.at[pslot]).wait()

        pltpu.async_copy(table.at[colv.at[j + NBUF - 1]], buf.at[pslot],
                         gsem.at[pslot])

      pltpu.make_async_copy(table.at[colv.at[j]], buf.at[slot],
                            gsem.at[slot]).wait()
      pltpu.async_copy(buf.at[slot], acc.at[rowv.at[j]], ssem.at[slot],
                       add=True)
      return carry

    lax.fori_loop(0, NCHUNK, body, 0)
    for p in range(NBUF):
      last = NCHUNK - NBUF + p
      pltpu.make_async_copy(
          buf.at[last % NBUF], acc.at[rowv.at[last]],
          ssem.at[last % NBUF]).wait()
    plsc.subcore_barrier()
    pltpu.sync_copy(acc.at[pl.ds(s * OPT, OPT)], out.at[c, pl.ds(s * OPT, OPT)])

  return agg


def _make_h0(interpret=False):
  RB = 512

  def body(su_ref, u_ref, b_ref, wt_ref, h0_ref, t_ref):
    su = su_ref[0]
    cnt = su[:, DH:DH + 1]
    agg = su[:, :DH] / jnp.maximum(cnt, 1.0)
    pre = agg + u_ref[0] + b_ref[0]
    nrm = jnp.sqrt(jnp.sum(pre * pre, axis=1, keepdims=True))
    h = jnp.tanh(pre / jnp.maximum(nrm, 1e-12))
    h0_ref[...] = h[None]
    t_ref[...] = jnp.dot(h, wt_ref[0], preferred_element_type=jnp.float32,
                         precision=_PREC)

  grid = (N // RB, 2)
  return pl.pallas_call(
      body,
      grid=grid,
      in_specs=[
          pl.BlockSpec((1, RB, 2 * DH), lambda i, j: (j, i, 0)),
          pl.BlockSpec((1, RB, DH), lambda i, j: (j, i, 0)),
          pl.BlockSpec((1, 1, DH), lambda i, j: (j, 0, 0)),
          pl.BlockSpec((1, DH, 2 * DH), lambda i, j: (j, 0, 0)),
      ],
      out_specs=[
          pl.BlockSpec((1, RB, DH), lambda i, j: (j, i, 0)),
          pl.BlockSpec((RB, 2 * DH), lambda i, j: (j * (N // RB) + i, 0)),
      ],
      out_shape=[
          jax.ShapeDtypeStruct((2, N, DH), jnp.float32),
          jax.ShapeDtypeStruct((2 * N, 2 * DH), jnp.float32),
      ],
      interpret=interpret,
  )


def _make_deep(interpret=False):
  RB = 512

  def body(sd_ref, tp_ref, tn_ref, h0_ref, cnt_ref, ws_ref, bd_ref, out_ref):
    ts = [tp_ref[...], tn_ref[...]]
    hs = []
    for sgn in range(2):
      oth = 1 - sgn
      c_own = cnt_ref[sgn][:, None] + 1.0
      c_oth = cnt_ref[oth][:, None] + 1.0
      own = (sd_ref[sgn][:, :DH] + ts[sgn][:, :DH]) / c_own
      cross = (sd_ref[oth][:, DH:] + ts[oth][:, DH:]) / c_oth
      slf = jnp.dot(h0_ref[sgn], ws_ref[sgn],
                    preferred_element_type=jnp.float32, precision=_PREC)
      pre = own + cross + slf + bd_ref[sgn][None, :]
      nrm = jnp.sqrt(jnp.sum(pre * pre, axis=1, keepdims=True))
      hs.append(jnp.tanh(pre / jnp.maximum(nrm, 1e-12)))
    z = jnp.concatenate(hs, axis=1)
    nrm = jnp.sqrt(jnp.sum(z * z, axis=1, keepdims=True))
    out_ref[...] = z / jnp.maximum(nrm, 1e-12)

  grid = (N // RB,)
  return pl.pallas_call(
      body,
      grid=grid,
      in_specs=[
          pl.BlockSpec((2, RB, 2 * DH), lambda i: (0, i, 0)),
          pl.BlockSpec((RB, 2 * DH), lambda i: (i, 0)),
          pl.BlockSpec((RB, 2 * DH), lambda i: (N // RB + i, 0)),
          pl.BlockSpec((2, RB, DH), lambda i: (0, i, 0)),
          pl.BlockSpec((2, RB), lambda i: (0, i)),
          pl.BlockSpec((2, DH, DH), lambda i: (0, 0, 0)),
          pl.BlockSpec((2, DH), lambda i: (0, 0)),
      ],
      out_specs=pl.BlockSpec((RB, 2 * DH), lambda i: (i, 0)),
      out_shape=jax.ShapeDtypeStruct((N, 2 * DH), jnp.float32),
      interpret=interpret,
  )


def _make_sim(interpret=False):
  BM, BN = 512, 4096
  GM, GN = N // BM, N // BN

  def body(a_ref, b_ref, m_ref, l_ref, pred_ref, ls_ref):
    i = pl.program_id(0)
    j = pl.program_id(1)
    g = lax.dot_general(a_ref[...], b_ref[...], (((1,), (1,)), ((), ())),
                        preferred_element_type=jnp.float32,
                        precision=_PREC)
    p = g * m_ref[...]
    pred_ref[...] = p.reshape(BM, BN // 128, 128)
    d = p - l_ref[...]
    part = jnp.sum(d * d)

    @pl.when(jnp.logical_and(i == 0, j == 0))
    def _():
      ls_ref[0, 0] = 0.0

    ls_ref[0, 0] += part

    @pl.when(jnp.logical_and(i == GM - 1, j == GN - 1))
    def _():
      ls_ref[0, 0] = ls_ref[0, 0] / (N * N)

  return pl.pallas_call(
      body,
      grid=(GM, GN),
      in_specs=[
          pl.BlockSpec((BM, 2 * DH), lambda i, j: (i, 0)),
          pl.BlockSpec((BN, 2 * DH), lambda i, j: (j, 0)),
          pl.BlockSpec((BM, BN), lambda i, j: (i, j)),
          pl.BlockSpec((BM, BN), lambda i, j: (i, j)),
      ],
      out_specs=[
          pl.BlockSpec((BM, BN // 128, 128), lambda i, j: (i, j, 0)),
          pl.BlockSpec((1, 1), lambda i, j: (0, 0),
                       memory_space=pltpu.SMEM),
      ],
      out_shape=[
          jax.ShapeDtypeStruct((N, N // 128, 128), jnp.float32),
          jax.ShapeDtypeStruct((1, 1), jnp.float32),
      ],
      interpret=interpret,
  )


_proj = _make_proj()
_h0 = _make_h0()
_deep = _make_deep()
_sim = _make_sim()


def kernel(X, W_pos_base, b_pos_base, W_neg_base, b_neg_base, W_pos_deep,
           b_pos_deep, W_neg_deep, b_neg_deep, labels, label_mask,
           positive_edges, negative_edges):
  Wy = jnp.stack([W_pos_base[:D_IN], W_neg_base[:D_IN]])
  Wu = jnp.stack([W_pos_base[D_IN:], W_neg_base[D_IN:]])
  bcat = jnp.stack([b_pos_base, b_neg_base])
  Wt = jnp.stack([
      jnp.concatenate([W_pos_deep[:DH], W_neg_deep[DH:2 * DH]], axis=1),
      jnp.concatenate([W_neg_deep[:DH], W_pos_deep[DH:2 * DH]], axis=1),
  ])
  Wself = jnp.stack([W_pos_deep[2 * DH:], W_neg_deep[2 * DH:]])
  bd = jnp.stack([b_pos_deep, b_neg_deep])
  edges_all = jnp.stack([positive_edges, negative_edges]).astype(jnp.int32)

  table_base, U, rows_adj, cols_adj = _proj(X, Wy, Wu, edges_all)
  cols_r = cols_adj.reshape(2, NS, NCHUNK, C)
  rows_r = rows_adj.reshape(2, NS, NCHUNK, C)

  zz = jnp.zeros((R, 2 * DH), jnp.float32)
  agg = _make_agg(2 * DH)
  s_base = agg(table_base, cols_r, rows_r, zz)
  cnt = s_base[:, :, DH]

  h0, t_deep = _h0(s_base, U, bcat[:, None, :], Wt)

  s_deep = agg(t_deep, cols_r, rows_r, zz)

  x_mol = _deep(s_deep, t_deep, t_deep, h0, cnt, Wself, bd)

  pred2d, lsum = _sim(x_mol, x_mol, label_mask, labels.reshape(N, N))
  return (lsum[0, 0], x_mol, pred2d.reshape(-1))


# speedup vs baseline: 11.4456x; 1.0053x over previous
import functools

import jax
import jax.numpy as jnp
from jax import lax
from jax.experimental import pallas as pl
from jax.experimental.pallas import tpu as pltpu
from jax.experimental.pallas import tpu_sc as plsc

N = 4096
E = 131072
D_IN = 300
DH = 64

NC = 2
NS = 16
C = 128
NBUF = 4
EPT = E // NS
NCHUNK = EPT // C
RPT = 264
R = NS * RPT
TRASH = N
OPT = N // NS

_PREC = lax.Precision.DEFAULT


def _make_proj(interpret=False):
  RB = 512

  def body(x_ref, wy_ref, wu_ref, e_ref, tbl_ref, u_ref, rows_ref, cols_ref):
    x = x_ref[...]
    y = jnp.dot(x, wy_ref[0], preferred_element_type=jnp.float32,
                precision=_PREC)
    u = jnp.dot(x, wu_ref[0], preferred_element_type=jnp.float32,
                precision=_PREC)
    ones = jnp.ones((RB, 16), jnp.float32)
    pad = jnp.zeros((RB, 48), jnp.float32)
    tbl_ref[...] = jnp.concatenate([y, ones, pad], axis=1)
    u_ref[...] = u[None]

    @pl.when(jnp.logical_and(pl.program_id(0) == 0, pl.program_id(1) == 0))
    def _():
      rows = e_ref[:, 0, :]
      cols = e_ref[:, 1, :]
      valid = rows != cols
      rows_ref[...] = jnp.where(valid, rows, TRASH)
      off = lax.broadcasted_iota(jnp.int32, (2, E), 0) * N
      cols_ref[...] = cols + off

  grid = (N // RB, 2)
  return pl.pallas_call(
      body,
      grid=grid,
      in_specs=[
          pl.BlockSpec((RB, D_IN), lambda i, j: (i, 0)),
          pl.BlockSpec((1, D_IN, DH), lambda i, j: (j, 0, 0)),
          pl.BlockSpec((1, D_IN, DH), lambda i, j: (j, 0, 0)),
          pl.BlockSpec((2, 2, E), lambda i, j: (0, 0, 0)),
      ],
      out_specs=[
          pl.BlockSpec((RB, 2 * DH), lambda i, j: (j * (N // RB) + i, 0)),
          pl.BlockSpec((1, RB, DH), lambda i, j: (j, i, 0)),
          pl.BlockSpec((2, E), lambda i, j: (0, 0)),
          pl.BlockSpec((2, E), lambda i, j: (0, 0)),
      ],
      out_shape=[
          jax.ShapeDtypeStruct((2 * N, 2 * DH), jnp.float32),
          jax.ShapeDtypeStruct((2, N, DH), jnp.float32),
          jax.ShapeDtypeStruct((2, E), jnp.int32),
          jax.ShapeDtypeStruct((2, E), jnp.int32),
      ],
      interpret=interpret,
  )


@functools.lru_cache(maxsize=None)
def _make_agg(width, interpret=False):
  mesh = plsc.VectorSubcoreMesh(core_axis_name="c", subcore_axis_name="s",
                                num_cores=NC, num_subcores=NS)

  @functools.partial(
      pl.kernel,
      out_type=jax.ShapeDtypeStruct((2, N, width), jnp.float32),
      mesh=mesh,
      scratch_types=[
          pltpu.VMEM((NCHUNK, C), jnp.int32),
          pltpu.VMEM((NCHUNK, C), jnp.int32),
          pltpu.VMEM((NBUF, C, width), jnp.float32),
          pltpu.VMEM_SHARED((R, width), jnp.float32),
          pltpu.SemaphoreType.DMA((NBUF,)),
          pltpu.SemaphoreType.DMA((NBUF,)),
      ],
      interpret=interpret,
  )
  def agg(table, cols, rows, zz, out, colv, rowv, buf, acc, gsem, ssem):
    c = lax.axis_index("c")
    s = lax.axis_index("s")
    pltpu.sync_copy(zz.at[pl.ds(s * RPT, RPT)], acc.at[pl.ds(s * RPT, RPT)])
    pltpu.sync_copy(cols.at[c, s], colv)
    pltpu.sync_copy(rows.at[c, s], rowv)
    plsc.subcore_barrier()

    for p in range(NBUF - 1):
      pltpu.async_copy(table.at[colv.at[p]], buf.at[p], gsem.at[p])

    def body(j, carry):
      slot = lax.rem(j, NBUF)
      pslot = lax.rem(j + NBUF - 1, NBUF)

      @pl.when(j + NBUF - 1 < NCHUNK)
      def _():
        @pl.when(j >= 1)
        def _():
          pltpu.make_async_copy(
              buf.at[pslot], acc.at[rowv.at[j - 1]], ssem.at[pslot]).wait()

        pltpu.async_copy(table.at[colv.at[j + NBUF - 1]], buf.at[pslot],
                         gsem.at[pslot])

      pltpu.make_async_copy(table.at[colv.at[j]], buf.at[slot],
                            gsem.at[slot]).wait()
      pltpu.async_copy(buf.at[slot], acc.at[rowv.at[j]], ssem.at[slot],
                       add=True)
      return carry

    lax.fori_loop(0, NCHUNK, body, 0)
    for p in range(NBUF):
      last = NCHUNK - NBUF + p
      pltpu.make_async_copy(
          buf.at[last % NBUF], acc.at[rowv.at[last]],
          ssem.at[last % NBUF]).wait()
    plsc.subcore_barrier()
    pltpu.sync_copy(acc.at[pl.ds(s * OPT, OPT)], out.at[c, pl.ds(s * OPT, OPT)])

  return agg


def _make_h0(interpret=False):
  RB = 512

  def body(su_ref, u_ref, b_ref, wt_ref, h0_ref, t_ref):
    su = su_ref[0]
    cnt = su[:, DH:DH + 1]
    agg = su[:, :DH] / jnp.maximum(cnt, 1.0)
    pre = agg + u_ref[0] + b_ref[0]
    nrm = jnp.sqrt(jnp.sum(pre * pre, axis=1, keepdims=True))
    h = jnp.tanh(pre / jnp.maximum(nrm, 1e-12))
    h0_ref[...] = h[None]
    t_ref[...] = jnp.dot(h, wt_ref[0], preferred_element_type=jnp.float32,
                         precision=_PREC)

  grid = (N // RB, 2)
  return pl.pallas_call(
      body,
      grid=grid,
      in_specs=[
          pl.BlockSpec((1, RB, 2 * DH), lambda i, j: (j, i, 0)),
          pl.BlockSpec((1, RB, DH), lambda i, j: (j, i, 0)),
          pl.BlockSpec((1, 1, DH), lambda i, j: (j, 0, 0)),
          pl.BlockSpec((1, DH, 2 * DH), lambda i, j: (j, 0, 0)),
      ],
      out_specs=[
          pl.BlockSpec((1, RB, DH), lambda i, j: (j, i, 0)),
          pl.BlockSpec((RB, 2 * DH), lambda i, j: (j * (N // RB) + i, 0)),
      ],
      out_shape=[
          jax.ShapeDtypeStruct((2, N, DH), jnp.float32),
          jax.ShapeDtypeStruct((2 * N, 2 * DH), jnp.float32),
      ],
      interpret=interpret,
  )


def _make_deep(interpret=False):
  RB = 512

  def body(sd_ref, tp_ref, tn_ref, h0_ref, cnt_ref, ws_ref, bd_ref, out_ref):
    ts = [tp_ref[...], tn_ref[...]]
    hs = []
    for sgn in range(2):
      oth = 1 - sgn
      c_own = cnt_ref[sgn][:, None] + 1.0
      c_oth = cnt_ref[oth][:, None] + 1.0
      own = (sd_ref[sgn][:, :DH] + ts[sgn][:, :DH]) / c_own
      cross = (sd_ref[oth][:, DH:] + ts[oth][:, DH:]) / c_oth
      slf = jnp.dot(h0_ref[sgn], ws_ref[sgn],
                    preferred_element_type=jnp.float32, precision=_PREC)
      pre = own + cross + slf + bd_ref[sgn][None, :]
      nrm = jnp.sqrt(jnp.sum(pre * pre, axis=1, keepdims=True))
      hs.append(jnp.tanh(pre / jnp.maximum(nrm, 1e-12)))
    z = jnp.concatenate(hs, axis=1)
    nrm = jnp.sqrt(jnp.sum(z * z, axis=1, keepdims=True))
    out_ref[...] = z / jnp.maximum(nrm, 1e-12)

  grid = (N // RB,)
  return pl.pallas_call(
      body,
      grid=grid,
      in_specs=[
          pl.BlockSpec((2, RB, 2 * DH), lambda i: (0, i, 0)),
          pl.BlockSpec((RB, 2 * DH), lambda i: (i, 0)),
          pl.BlockSpec((RB, 2 * DH), lambda i: (N // RB + i, 0)),
          pl.BlockSpec((2, RB, DH), lambda i: (0, i, 0)),
          pl.BlockSpec((2, RB), lambda i: (0, i)),
          pl.BlockSpec((2, DH, DH), lambda i: (0, 0, 0)),
          pl.BlockSpec((2, DH), lambda i: (0, 0)),
      ],
      out_specs=pl.BlockSpec((RB, 2 * DH), lambda i: (i, 0)),
      out_shape=jax.ShapeDtypeStruct((N, 2 * DH), jnp.float32),
      interpret=interpret,
  )


def _make_sim(interpret=False):
  BM, BN = 256, 4096
  GM, GN = N // BM, N // BN

  def body(a_ref, b_ref, m_ref, l_ref, pred_ref, ls_ref):
    i = pl.program_id(0)
    j = pl.program_id(1)
    g = lax.dot_general(a_ref[...], b_ref[...], (((1,), (1,)), ((), ())),
                        preferred_element_type=jnp.float32,
                        precision=_PREC)
    p = g * m_ref[...]
    pred_ref[...] = p.reshape(BM, BN // 128, 128)
    d = p - l_ref[...]
    part = jnp.sum(d * d)

    @pl.when(jnp.logical_and(i == 0, j == 0))
    def _():
      ls_ref[0, 0] = 0.0

    ls_ref[0, 0] += part

    @pl.when(jnp.logical_and(i == GM - 1, j == GN - 1))
    def _():
      ls_ref[0, 0] = ls_ref[0, 0] / (N * N)

  return pl.pallas_call(
      body,
      grid=(GM, GN),
      in_specs=[
          pl.BlockSpec((BM, 2 * DH), lambda i, j: (i, 0)),
          pl.BlockSpec((BN, 2 * DH), lambda i, j: (j, 0)),
          pl.BlockSpec((BM, BN), lambda i, j: (i, j)),
          pl.BlockSpec((BM, BN), lambda i, j: (i, j)),
      ],
      out_specs=[
          pl.BlockSpec((BM, BN // 128, 128), lambda i, j: (i, j, 0)),
          pl.BlockSpec((1, 1), lambda i, j: (0, 0),
                       memory_space=pltpu.SMEM),
      ],
      out_shape=[
          jax.ShapeDtypeStruct((N, N // 128, 128), jnp.float32),
          jax.ShapeDtypeStruct((1, 1), jnp.float32),
      ],
      interpret=interpret,
  )


_proj = _make_proj()
_h0 = _make_h0()
_deep = _make_deep()
_sim = _make_sim()


def kernel(X, W_pos_base, b_pos_base, W_neg_base, b_neg_base, W_pos_deep,
           b_pos_deep, W_neg_deep, b_neg_deep, labels, label_mask,
           positive_edges, negative_edges):
  Wy = jnp.stack([W_pos_base[:D_IN], W_neg_base[:D_IN]])
  Wu = jnp.stack([W_pos_base[D_IN:], W_neg_base[D_IN:]])
  bcat = jnp.stack([b_pos_base, b_neg_base])
  Wt = jnp.stack([
      jnp.concatenate([W_pos_deep[:DH], W_neg_deep[DH:2 * DH]], axis=1),
      jnp.concatenate([W_neg_deep[:DH], W_pos_deep[DH:2 * DH]], axis=1),
  ])
  Wself = jnp.stack([W_pos_deep[2 * DH:], W_neg_deep[2 * DH:]])
  bd = jnp.stack([b_pos_deep, b_neg_deep])
  edges_all = jnp.stack([positive_edges, negative_edges]).astype(jnp.int32)

  table_base, U, rows_adj, cols_adj = _proj(X, Wy, Wu, edges_all)
  cols_r = cols_adj.reshape(2, NS, NCHUNK, C)
  rows_r = rows_adj.reshape(2, NS, NCHUNK, C)

  zz = jnp.zeros((R, 2 * DH), jnp.float32)
  agg = _make_agg(2 * DH)
  s_base = agg(table_base, cols_r, rows_r, zz)
  cnt = s_base[:, :, DH]

  h0, t_deep = _h0(s_base, U, bcat[:, None, :], Wt)

  s_deep = agg(t_deep, cols_r, rows_r, zz)

  x_mol = _deep(s_deep, t_deep, t_deep, h0, cnt, Wself, bd)

  pred2d, lsum = _sim(x_mol, x_mol, label_mask, labels.reshape(N, N))
  return (lsum[0, 0], x_mol, pred2d.reshape(-1))
